# baseline (speedup 1.0000x reference)
import functools

import jax
import jax.numpy as jnp
from jax import lax
from jax.experimental import pallas as pl
from jax.experimental.pallas import tpu as pltpu
from jax.experimental.pallas import tpu_sc as plsc

_NUM_EMBEDDINGS = 100000
_DIM = 128
_BATCH = 4096
_HIST = 50

_INFO = plsc.get_sparse_core_info()
_NC = _INFO.num_cores
_NS = _INFO.num_subcores
_NW = _NC * _NS

_K = 4
_EPC = _BATCH // _K
_RPC = _EPC * _HIST
_RPW = _RPC // _NW
_CH = 80
_NCHUNK = _RPW // _CH
_NBUF = 4
_LOOKAHEAD = 2

_EPB = 8
_TC_GRID = _EPC // _EPB


def _gather_body(idx_hbm, table_hbm, out_hbm, idx_v, rows_v, gsems, ssems):
    wid = lax.axis_index("s") * _NC + lax.axis_index("c")
    base = wid * _RPW

    pltpu.sync_copy(idx_hbm.at[wid], idx_v)

    def start_gather(c, b):
        pltpu.async_copy(table_hbm.at[idx_v.at[c]], rows_v.at[b], gsems.at[b])

    def wait_gather(b):
        pltpu.make_async_copy(
            table_hbm.at[idx_v.at[0]], rows_v.at[b], gsems.at[b]
        ).wait()

    def start_store(c, b):
        pltpu.async_copy(
            rows_v.at[b], out_hbm.at[pl.ds(base + c * _CH, _CH)], ssems.at[b]
        )

    def wait_store(b):
        pltpu.make_async_copy(
            rows_v.at[b], out_hbm.at[pl.ds(0, _CH)], ssems.at[b]
        ).wait()

    for c0 in range(_LOOKAHEAD):
        start_gather(c0, c0)

    def group_body(g, carry):
        del carry
        for b in range(_NBUF):
            c = g * _NBUF + b
            tb = (b + _LOOKAHEAD) % _NBUF

            @pl.when(c + _LOOKAHEAD < _NCHUNK)
            def _():
                @pl.when(c + _LOOKAHEAD >= _NBUF)
                def _():
                    wait_store(tb)

                start_gather(c + _LOOKAHEAD, tb)

            wait_gather(b)
            start_store(c, b)
        return 0

    lax.fori_loop(0, _NCHUNK // _NBUF, group_body, 0)

    for b in range(_NBUF):
        wait_store(b)


def _relayout_first_body(flat_ref, out_ref):
    for e in range(_EPB):
        out_ref[e] = flat_ref[pl.ds(e * _HIST, _HIST), :]


def _relayout_body(flat_ref, acc_ref, out_ref):
    del acc_ref
    _relayout_first_body(flat_ref, out_ref)


def _relayout(k, flat, acc):
    flat_spec = pl.BlockSpec((_EPB * _HIST, _DIM), lambda i: (i, 0))
    out_spec = pl.BlockSpec(
        (_EPB, _HIST, _DIM), lambda i, _k=k: (_k * _TC_GRID + i, 0, 0)
    )
    out_shape = jax.ShapeDtypeStruct((_BATCH, _HIST, _DIM), jnp.float32)
    if acc is None:
        return pl.pallas_call(
            _relayout_first_body,
            grid=(_TC_GRID,),
            in_specs=[flat_spec],
            out_specs=out_spec,
            out_shape=out_shape,
        )(flat)
    return pl.pallas_call(
        _relayout_body,
        grid=(_TC_GRID,),
        in_specs=[flat_spec, pl.BlockSpec(memory_space=pl.ANY)],
        out_specs=out_spec,
        out_shape=out_shape,
        input_output_aliases={1: 0},
    )(flat, acc)


@jax.jit
def kernel(x, table):
    idx = x.reshape(_K, _NW, _NCHUNK, _CH).astype(jnp.int32)
    call = pl.kernel(
        _gather_body,
        out_type=jax.ShapeDtypeStruct((_RPC, _DIM), jnp.float32),
        mesh=plsc.VectorSubcoreMesh(core_axis_name="c", subcore_axis_name="s"),
        scratch_types=[
            pltpu.VMEM((_NCHUNK, _CH), jnp.int32),
            pltpu.VMEM((_NBUF, _CH, _DIM), jnp.float32),
            pltpu.SemaphoreType.DMA((_NBUF,)),
            pltpu.SemaphoreType.DMA((_NBUF,)),
        ],
    )
    acc = None
    for k in range(_K):
        flat = call(idx[k], table)
        acc = _relayout(k, flat, acc)
    return acc
